# baseline (speedup 1.0000x reference)
import functools

import jax
import jax.numpy as jnp
from jax.experimental import pallas as pl
from jax.experimental.pallas import tpu as pltpu

_NE = 1024
_D = 64
_R = 1024


def _vq_block(xt_ref, w_ref, wt_ref, qt_ref, counts_ref, sse_ref):
    xt = xt_ref[0]
    xb = jnp.transpose(xt, (1, 0))
    wt = wt_ref[...]
    x2 = jnp.sum(xb * xb, axis=1, keepdims=True)
    w2 = jnp.sum(wt * wt, axis=0, keepdims=True)
    mm = jax.lax.dot_general(xb, wt, (((1,), (0,)), ((), ())),
                             preferred_element_type=jnp.float32)
    d = (x2 + w2) - 2.0 * mm
    lane = jax.lax.broadcasted_iota(jnp.int32, d.shape, 1)
    dmin = jnp.min(d, axis=1, keepdims=True)
    idx = jnp.min(jnp.where(d == dmin, lane, _NE), axis=1, keepdims=True)
    onehot = (lane == idx).astype(jnp.float32)
    qt = jax.lax.dot_general(wt_ref[...], onehot, (((1,), (1,)), ((), ())),
                             preferred_element_type=jnp.float32)
    qt_ref[0] = qt
    diff = qt - xt
    cb = jnp.sum(onehot, axis=0, keepdims=True)
    sb = jnp.sum(jnp.sum(diff * diff, axis=1, keepdims=True),
                 axis=0, keepdims=True)

    @pl.when(pl.program_id(0) == 0)
    def _init():
        counts_ref[...] = cb
        sse_ref[...] = sb

    @pl.when(pl.program_id(0) != 0)
    def _acc():
        counts_ref[...] += cb
        sse_ref[...] += sb


@functools.partial(jax.jit, static_argnames=())
def kernel(x, W):
    B, C, H, Wd = x.shape
    n = B * H * Wd
    xt = x.reshape(B, C, H * Wd)
    wt = W.T
    grid = n // _R
    qt, counts, sse = pl.pallas_call(
        _vq_block,
        grid=(grid,),
        in_specs=[
            pl.BlockSpec((1, _D, _R), lambda i: (i, 0, 0)),
            pl.BlockSpec((_NE, _D), lambda i: (0, 0)),
            pl.BlockSpec((_D, _NE), lambda i: (0, 0)),
        ],
        out_specs=[
            pl.BlockSpec((1, _D, _R), lambda i: (i, 0, 0)),
            pl.BlockSpec((1, _NE), lambda i: (0, 0)),
            pl.BlockSpec((1, 1), lambda i: (0, 0)),
        ],
        out_shape=[
            jax.ShapeDtypeStruct((B, C, H * Wd), jnp.float32),
            jax.ShapeDtypeStruct((1, _NE), jnp.float32),
            jax.ShapeDtypeStruct((1, 1), jnp.float32),
        ],
        compiler_params=pltpu.CompilerParams(
            dimension_semantics=("arbitrary",),
        ),
    )(xt, W, wt)
    quantized = qt.reshape(B, C, H, Wd)
    m = sse[0, 0] / (n * _D)
    loss = m + 0.25 * m
    avg_probs = counts[0] / n
    perplexity = jnp.exp(-jnp.sum(avg_probs * jnp.log(avg_probs + 1e-10)))
    return (quantized, loss, perplexity)


# speedup vs baseline: 1.2868x; 1.2868x over previous
import functools

import jax
import jax.numpy as jnp
from jax.experimental import pallas as pl
from jax.experimental.pallas import tpu as pltpu

_NE = 1024
_D = 64
_R = 2048


def _vq_block(x_ref, w_ref, wt_ref, q_ref, counts_ref, sse_ref):
    xb = x_ref[...]
    wt = wt_ref[...]
    x2 = jnp.sum(xb * xb, axis=1, keepdims=True)
    w2 = jnp.sum(wt * wt, axis=0, keepdims=True)
    mm = jax.lax.dot_general(xb, wt, (((1,), (0,)), ((), ())),
                             preferred_element_type=jnp.float32)
    d = (x2 + w2) - 2.0 * mm
    lane = jax.lax.broadcasted_iota(jnp.int32, d.shape, 1)
    dmin = jnp.min(d, axis=1, keepdims=True)
    idx = jnp.min(jnp.where(d == dmin, lane, _NE), axis=1, keepdims=True)
    onehot = (lane == idx).astype(jnp.float32)
    qb = jax.lax.dot_general(onehot, w_ref[...], (((1,), (0,)), ((), ())),
                             preferred_element_type=jnp.float32)
    q_ref[...] = qb
    diff = qb - xb
    cb = jnp.sum(onehot, axis=0, keepdims=True)
    sb = jnp.sum(jnp.sum(diff * diff, axis=1, keepdims=True),
                 axis=0, keepdims=True)

    @pl.when(pl.program_id(0) == 0)
    def _init():
        counts_ref[...] = cb
        sse_ref[...] = sb

    @pl.when(pl.program_id(0) != 0)
    def _acc():
        counts_ref[...] += cb
        sse_ref[...] += sb


@functools.partial(jax.jit, static_argnames=())
def kernel(x, W):
    B, C, H, Wd = x.shape
    n = B * H * Wd
    x_flat = jnp.transpose(x, (0, 2, 3, 1)).reshape(n, _D)
    wt = W.T
    grid = n // _R
    q, counts, sse = pl.pallas_call(
        _vq_block,
        grid=(grid,),
        in_specs=[
            pl.BlockSpec((_R, _D), lambda i: (i, 0)),
            pl.BlockSpec((_NE, _D), lambda i: (0, 0)),
            pl.BlockSpec((_D, _NE), lambda i: (0, 0)),
        ],
        out_specs=[
            pl.BlockSpec((_R, _D), lambda i: (i, 0)),
            pl.BlockSpec((1, _NE), lambda i: (0, 0)),
            pl.BlockSpec((1, 1), lambda i: (0, 0)),
        ],
        out_shape=[
            jax.ShapeDtypeStruct((n, _D), jnp.float32),
            jax.ShapeDtypeStruct((1, _NE), jnp.float32),
            jax.ShapeDtypeStruct((1, 1), jnp.float32),
        ],
        compiler_params=pltpu.CompilerParams(
            dimension_semantics=("arbitrary",),
        ),
    )(x_flat, W, wt)
    quantized = q.reshape(B, H, Wd, C).transpose(0, 3, 1, 2)
    m = sse[0, 0] / (n * _D)
    loss = m + 0.25 * m
    avg_probs = counts[0] / n
    perplexity = jnp.exp(-jnp.sum(avg_probs * jnp.log(avg_probs + 1e-10)))
    return (quantized, loss, perplexity)


# speedup vs baseline: 1.3768x; 1.0699x over previous
import functools

import jax
import jax.numpy as jnp
from jax.experimental import pallas as pl
from jax.experimental.pallas import tpu as pltpu

_NE = 1024
_D = 64
_R = 2048


def _vq_block(x_ref, w_ref, wtm2_ref, iota_ref, q_ref, counts_ref, sse_ref):
    xb = x_ref[...]
    wtm2 = wtm2_ref[...]
    x2 = jnp.sum(xb * xb, axis=1, keepdims=True)
    w2 = 0.25 * jnp.sum(wtm2 * wtm2, axis=0, keepdims=True)
    mm2 = jax.lax.dot_general(xb, wtm2, (((1,), (0,)), ((), ())),
                              preferred_element_type=jnp.float32)
    d = (x2 + w2) + mm2
    lane = iota_ref[...]
    dmin = jnp.min(d, axis=1, keepdims=True)
    idx = jnp.min(jnp.where(d == dmin, lane, float(_NE)), axis=1,
                  keepdims=True)
    onehot = (lane == idx).astype(jnp.float32)
    qb = jax.lax.dot_general(onehot, w_ref[...], (((1,), (0,)), ((), ())),
                             preferred_element_type=jnp.float32)
    q_ref[...] = qb
    cb = jnp.sum(onehot, axis=0, keepdims=True)
    sb = jnp.sum(dmin, axis=0, keepdims=True)

    @pl.when(pl.program_id(0) == 0)
    def _init():
        counts_ref[...] = cb
        sse_ref[...] = sb

    @pl.when(pl.program_id(0) != 0)
    def _acc():
        counts_ref[...] += cb
        sse_ref[...] += sb


@functools.partial(jax.jit, static_argnames=())
def kernel(x, W):
    B, C, H, Wd = x.shape
    n = B * H * Wd
    x_flat = jnp.transpose(x, (0, 2, 3, 1)).reshape(n, _D)
    wtm2 = -2.0 * W.T
    grid = n // _R
    q, counts, sse = pl.pallas_call(
        _vq_block,
        grid=(grid,),
        in_specs=[
            pl.BlockSpec((_R, _D), lambda i: (i, 0)),
            pl.BlockSpec((_NE, _D), lambda i: (0, 0)),
            pl.BlockSpec((_D, _NE), lambda i: (0, 0)),
            pl.BlockSpec((1, _NE), lambda i: (0, 0)),
        ],
        out_specs=[
            pl.BlockSpec((_R, _D), lambda i: (i, 0)),
            pl.BlockSpec((1, _NE), lambda i: (0, 0)),
            pl.BlockSpec((1, 1), lambda i: (0, 0)),
        ],
        out_shape=[
            jax.ShapeDtypeStruct((n, _D), jnp.float32),
            jax.ShapeDtypeStruct((1, _NE), jnp.float32),
            jax.ShapeDtypeStruct((1, 1), jnp.float32),
        ],
        compiler_params=pltpu.CompilerParams(
            dimension_semantics=("arbitrary",),
        ),
    )(x_flat, W, wtm2, jnp.arange(_NE, dtype=jnp.float32).reshape(1, _NE))
    quantized = q.reshape(B, H, Wd, C).transpose(0, 3, 1, 2)
    m = sse[0, 0] / (n * _D)
    loss = m + 0.25 * m
    avg_probs = counts[0] / n
    perplexity = jnp.exp(-jnp.sum(avg_probs * jnp.log(avg_probs + 1e-10)))
    return (quantized, loss, perplexity)
